# Initial kernel scaffold; baseline (speedup 1.0000x reference)
import jax
import jax.numpy as jnp
from jax.experimental import pallas as pl


def kernel(x, conv1_w, conv1_b, conv2_w, conv2_b, fc_w1, fc_b1, fc_w2, fc_b2, fc_w3, fc_b3):
    raise NotImplementedError("write your pallas kernel here")


# speedup vs baseline: 7.3909x; 7.3909x over previous
import numpy as np
import jax
import jax.numpy as jnp
from jax.experimental import pallas as pl
from jax.experimental.pallas import tpu as pltpu

LANES = 128
_TB = 256


def _w1_indices():
    row = np.full((128, 512), 9, np.int32)
    col = np.zeros((128, 512), np.int32)
    for r in range(4):
        for cp in range(30):
            k = r * 30 + cp
            for ph in range(4):
                di, dj = divmod(ph, 2)
                ki = r - di
                if not 0 <= ki < 3:
                    continue
                for q in range(14):
                    kj = cp - 2 * q - dj
                    if not 0 <= kj < 3:
                        continue
                    for c in range(6):
                        n = ph * 128 + (q + 1) * 6 + c
                        row[k, n] = ki * 3 + kj
                        col[k, n] = c
    return row, col


def _w2_indices():
    row = np.full((4, 128, 512), 54, np.int32)
    col = np.zeros((4, 128, 512), np.int32)
    for r in range(4):
        for cp in range(16):
            for ch in range(6):
                k = cp * 6 + ch
                for ph in range(4):
                    di, dj = divmod(ph, 2)
                    ki = r - di
                    if not 0 <= ki < 3:
                        continue
                    for q2 in range(7):
                        kj = cp - 2 * q2 - dj
                        if not 0 <= kj < 3:
                            continue
                        for o in range(16):
                            n = ph * 128 + q2 * 16 + o
                            row[r, k, n] = ki * 18 + kj * 6 + ch
                            col[r, k, n] = o
    return row, col


_ROW1, _COL1 = _w1_indices()
_ROW2, _COL2 = _w2_indices()
_IB1 = np.array([(n - 6) % 6 if 6 <= n < 90 else 6 for n in range(128)], np.int32)
_IB2 = np.array([n % 16 if n < 112 else 16 for n in range(128)], np.int32)


def _net_kernel(p1, w1, w2, f1, fw2, fw3, b1, b2, fb1, fb2, fb3, out):
    tb = out.shape[0]
    w1v = w1[...]
    b1v = b1[...]

    rows = [None] * 16
    zero = jnp.zeros((tb, 128), jnp.bfloat16)
    rows[0] = zero
    rows[15] = zero
    for po in range(14):
        v = jnp.dot(p1[po], w1v, preferred_element_type=jnp.float32)
        m = jnp.maximum(jnp.maximum(v[:, 0:128], v[:, 128:256]),
                        jnp.maximum(v[:, 256:384], v[:, 384:512]))
        rows[po + 1] = jnp.maximum(m + b1v, 0.0).astype(jnp.bfloat16)

    b2v = b2[...]
    h1 = jnp.zeros((tb, 128), jnp.float32)
    for po2 in range(7):
        acc = None
        for r in range(4):
            ri = 2 * po2 + r
            if ri in (0, 15):
                continue
            t = jnp.dot(rows[ri], w2[r], preferred_element_type=jnp.float32)
            acc = t if acc is None else acc + t
        m = jnp.maximum(jnp.maximum(acc[:, 0:128], acc[:, 128:256]),
                        jnp.maximum(acc[:, 256:384], acc[:, 384:512]))
        row2 = jnp.maximum(m + b2v, 0.0).astype(jnp.bfloat16)
        h1 = h1 + jnp.dot(row2, f1[po2], preferred_element_type=jnp.float32)

    h1 = jnp.maximum(h1 + fb1[...], 0.0).astype(jnp.bfloat16)
    h2 = jnp.maximum(
        jnp.dot(h1, fw2[...], preferred_element_type=jnp.float32) + fb2[...],
        0.0).astype(jnp.bfloat16)
    out[...] = jnp.dot(h2, fw3[...], preferred_element_type=jnp.float32) + fb3[...]


def kernel(x, conv1_w, conv1_b, conv2_w, conv2_b,
           fc_w1, fc_b1, fc_w2, fc_b2, fc_w3, fc_b3):
    B = x.shape[0]
    tb = _TB if B % _TB == 0 else B
    grid = B // tb

    w1e = jnp.concatenate([conv1_w, jnp.zeros((1, 128), conv1_w.dtype)], axis=0)
    W1 = w1e[_ROW1, _COL1]
    w2e = jnp.concatenate([conv2_w, jnp.zeros((1, 128), conv2_w.dtype)], axis=0)
    W2 = w2e[_ROW2, _COL2]
    F1 = jnp.pad(fc_w1.reshape(7, 112, 128), ((0, 0), (0, 16), (0, 0)))
    b1v = conv1_b[0][_IB1][None, :]
    b2v = conv2_b[0][_IB2][None, :]

    x3 = x.reshape(B, 28, 28).astype(jnp.bfloat16)
    xp = jnp.pad(x3, ((0, 0), (1, 1), (1, 1)))
    t = jnp.stack([xp[:, r:r + 28:2, :] for r in range(4)], axis=0)
    P1 = jnp.transpose(t, (2, 1, 0, 3)).reshape(14, B, 120)
    P1 = jnp.pad(P1, ((0, 0), (0, 0), (0, 8)))

    res = pl.pallas_call(
        _net_kernel,
        out_shape=jax.ShapeDtypeStruct((B, LANES), jnp.float32),
        grid=(grid,),
        in_specs=[
            pl.BlockSpec((14, tb, 128), lambda i: (0, i, 0)),
            pl.BlockSpec((128, 512), lambda i: (0, 0)),
            pl.BlockSpec((4, 128, 512), lambda i: (0, 0, 0)),
            pl.BlockSpec((7, 128, 128), lambda i: (0, 0, 0)),
            pl.BlockSpec((128, 128), lambda i: (0, 0)),
            pl.BlockSpec((128, 128), lambda i: (0, 0)),
            pl.BlockSpec((1, 128), lambda i: (0, 0)),
            pl.BlockSpec((1, 128), lambda i: (0, 0)),
            pl.BlockSpec((1, 128), lambda i: (0, 0)),
            pl.BlockSpec((1, 128), lambda i: (0, 0)),
            pl.BlockSpec((1, 128), lambda i: (0, 0)),
        ],
        out_specs=pl.BlockSpec((tb, LANES), lambda i: (i, 0)),
        compiler_params=pltpu.CompilerParams(
            dimension_semantics=("parallel",)),
    )(P1, W1, W2, F1, fc_w2, fc_w3, b1v, b2v, fc_b1, fc_b2, fc_b3)

    return res[:, :10]


# speedup vs baseline: 95.6437x; 12.9407x over previous
import jax
import jax.numpy as jnp
from jax.experimental import pallas as pl
from jax.experimental.pallas import tpu as pltpu

LANES = 128
_TB = 256


def _build_conv1_weights(conv1_w):
    dt = conv1_w.dtype
    f = conv1_w[:, :6]

    def z(n, m):
        return jnp.zeros((n, m), dt)

    phase_blocks = []
    for ph in range(4):
        di, dj = divmod(ph, 2)
        lead = di * 30 + dj
        E = jnp.concatenate(
            [z(lead, 6), f[0:3], z(27, 6), f[3:6], z(27, 6), f[6:9],
             z(128 - lead - 63, 6)], axis=0)
        cols = [z(128, 6)]
        for q in range(14):
            cols.append(jnp.concatenate([z(2 * q, 6), E[:128 - 2 * q]], axis=0))
        cols.append(z(128, 38))
        phase_blocks.append(jnp.concatenate(cols, axis=1))
    W1big = jnp.concatenate(phase_blocks, axis=1)

    W1r = W1big[:120].reshape(4, 30, 512)
    w1mid = W1r[:, 1:29].reshape(112, 512)
    w1top = W1r[1:4, 1:29].reshape(84, 512)
    w1bot = W1r[0:3, 1:29].reshape(84, 512)
    return w1top, w1mid, w1bot


def _build_conv2_weights(conv2_w):
    dt = conv2_w.dtype

    def z(n, m):
        return jnp.zeros((n, m), dt)

    per_r = []
    for r in range(4):
        phase_blocks = []
        for ph in range(4):
            di, dj = divmod(ph, 2)
            ki = r - di
            if 0 <= ki <= 2:
                sl = conv2_w[ki * 18:ki * 18 + 18, :16]
                E = jnp.concatenate([z(6 * dj, 16), sl, z(6 - 6 * dj, 16)],
                                    axis=0)
            else:
                E = z(24, 16)
            E128 = jnp.concatenate([E, z(104, 16)], axis=0)
            cols = []
            for q2 in range(7):
                cols.append(jnp.concatenate(
                    [z(12 * q2, 16), E128[:128 - 12 * q2]], axis=0))
            cols.append(z(128, 16))
            phase_blocks.append(jnp.concatenate(cols, axis=1))
        per_r.append(jnp.concatenate(phase_blocks, axis=1))
    return jnp.stack(per_r, axis=0)


def _net_kernel(x_ref, w1t, w1m, w1b, w2, f1, fw2, fw3,
                b1, b2, fb1, fb2, fb3, out):
    tb = out.shape[0]
    xv = x_ref[...].astype(jnp.bfloat16)
    b1v = b1[...]

    rows = [None] * 16
    zero = jnp.zeros((tb, 128), jnp.bfloat16)
    rows[0] = zero
    rows[15] = zero
    for po in range(14):
        if po == 0:
            v = jnp.dot(xv[:, 0:84], w1t[...],
                        preferred_element_type=jnp.float32)
        elif po == 13:
            v = jnp.dot(xv[:, 700:784], w1b[...],
                        preferred_element_type=jnp.float32)
        else:
            v = jnp.dot(xv[:, 56 * po - 28:56 * po + 84], w1m[...],
                        preferred_element_type=jnp.float32)
        m = jnp.maximum(jnp.maximum(v[:, 0:128], v[:, 128:256]),
                        jnp.maximum(v[:, 256:384], v[:, 384:512]))
        rows[po + 1] = jnp.maximum(m + b1v, 0.0).astype(jnp.bfloat16)

    b2v = b2[...]
    h1 = jnp.zeros((tb, 128), jnp.float32)
    for po2 in range(7):
        acc = None
        for r in range(4):
            ri = 2 * po2 + r
            if ri in (0, 15):
                continue
            t = jnp.dot(rows[ri], w2[r], preferred_element_type=jnp.float32)
            acc = t if acc is None else acc + t
        m = jnp.maximum(jnp.maximum(acc[:, 0:128], acc[:, 128:256]),
                        jnp.maximum(acc[:, 256:384], acc[:, 384:512]))
        row2 = jnp.maximum(m + b2v, 0.0).astype(jnp.bfloat16)
        h1 = h1 + jnp.dot(row2, f1[po2], preferred_element_type=jnp.float32)

    h1 = jnp.maximum(h1 + fb1[...], 0.0).astype(jnp.bfloat16)
    h2 = jnp.maximum(
        jnp.dot(h1, fw2[...], preferred_element_type=jnp.float32) + fb2[...],
        0.0).astype(jnp.bfloat16)
    out[...] = jnp.dot(h2, fw3[...], preferred_element_type=jnp.float32) + fb3[...]


def kernel(x, conv1_w, conv1_b, conv2_w, conv2_b,
           fc_w1, fc_b1, fc_w2, fc_b2, fc_w3, fc_b3):
    B = x.shape[0]
    tb = _TB if B % _TB == 0 else B
    grid = B // tb

    w1t, w1m, w1b = _build_conv1_weights(conv1_w)
    W2 = _build_conv2_weights(conv2_w)
    F1 = jnp.pad(fc_w1.reshape(7, 112, 128), ((0, 0), (0, 16), (0, 0)))
    b1v = jnp.concatenate([jnp.zeros((6,), jnp.float32),
                           jnp.tile(conv1_b[0, :6], 14),
                           jnp.zeros((38,), jnp.float32)])[None, :]
    b2v = jnp.concatenate([jnp.tile(conv2_b[0, :16], 7),
                           jnp.zeros((16,), jnp.float32)])[None, :]

    x2d = x.reshape(B, 784)

    res = pl.pallas_call(
        _net_kernel,
        out_shape=jax.ShapeDtypeStruct((B, LANES), jnp.float32),
        grid=(grid,),
        in_specs=[
            pl.BlockSpec((tb, 784), lambda i: (i, 0)),
            pl.BlockSpec((84, 512), lambda i: (0, 0)),
            pl.BlockSpec((112, 512), lambda i: (0, 0)),
            pl.BlockSpec((84, 512), lambda i: (0, 0)),
            pl.BlockSpec((4, 128, 512), lambda i: (0, 0, 0)),
            pl.BlockSpec((7, 128, 128), lambda i: (0, 0, 0)),
            pl.BlockSpec((128, 128), lambda i: (0, 0)),
            pl.BlockSpec((128, 128), lambda i: (0, 0)),
            pl.BlockSpec((1, 128), lambda i: (0, 0)),
            pl.BlockSpec((1, 128), lambda i: (0, 0)),
            pl.BlockSpec((1, 128), lambda i: (0, 0)),
            pl.BlockSpec((1, 128), lambda i: (0, 0)),
            pl.BlockSpec((1, 128), lambda i: (0, 0)),
        ],
        out_specs=pl.BlockSpec((tb, LANES), lambda i: (i, 0)),
        compiler_params=pltpu.CompilerParams(
            dimension_semantics=("parallel",)),
    )(x2d, w1t, w1m, w1b, W2, F1, fc_w2, fc_w3, b1v, b2v,
      fc_b1, fc_b2, fc_b3)

    return res[:, :10]


# speedup vs baseline: 103.5314x; 1.0825x over previous
import jax
import jax.numpy as jnp
from jax.experimental import pallas as pl
from jax.experimental.pallas import tpu as pltpu

LANES = 128
_TB = 512


def _build_conv1_weights(conv1_w):
    dt = conv1_w.dtype
    f = conv1_w[:, :6]

    def z(n, m):
        return jnp.zeros((n, m), dt)

    phase_blocks = []
    for ph in range(4):
        di, dj = divmod(ph, 2)
        lead = di * 30 + dj
        E = jnp.concatenate(
            [z(lead, 6), f[0:3], z(27, 6), f[3:6], z(27, 6), f[6:9],
             z(128 - lead - 63, 6)], axis=0)
        cols = [z(128, 6)]
        for q in range(14):
            cols.append(jnp.concatenate([z(2 * q, 6), E[:128 - 2 * q]], axis=0))
        cols.append(z(128, 38))
        phase_blocks.append(jnp.concatenate(cols, axis=1))
    W1big = jnp.concatenate(phase_blocks, axis=1)

    W1r = W1big[:120].reshape(4, 30, 512)
    w1mid = W1r[:, 1:29].reshape(112, 512)
    w1top = W1r[1:4, 1:29].reshape(84, 512)
    w1bot = W1r[0:3, 1:29].reshape(84, 512)
    return w1top, w1mid, w1bot


def _build_conv2_weights(conv2_w):
    dt = conv2_w.dtype

    def z(n, m):
        return jnp.zeros((n, m), dt)

    per_r = []
    for r in range(4):
        phase_blocks = []
        for ph in range(4):
            di, dj = divmod(ph, 2)
            ki = r - di
            if 0 <= ki <= 2:
                sl = conv2_w[ki * 18:ki * 18 + 18, :16]
                E = jnp.concatenate([z(6 * dj, 16), sl, z(6 - 6 * dj, 16)],
                                    axis=0)
            else:
                E = z(24, 16)
            E128 = jnp.concatenate([E, z(104, 16)], axis=0)
            cols = []
            for q2 in range(7):
                cols.append(jnp.concatenate(
                    [z(12 * q2, 16), E128[:128 - 12 * q2]], axis=0))
            cols.append(z(128, 16))
            phase_blocks.append(jnp.concatenate(cols, axis=1))
        per_r.append(jnp.concatenate(phase_blocks, axis=1))
    return jnp.stack(per_r, axis=0)


def _net_kernel(x_ref, w1t, w1m, w1b, w2, f1, fw2, fw3,
                b1, b2, fb1, fb2, fb3, out):
    tb = out.shape[0]
    xv = x_ref[...]
    b1v = b1[...]

    rows = [None] * 16
    zero = jnp.zeros((tb, 128), jnp.bfloat16)
    rows[0] = zero
    rows[15] = zero
    for po in range(14):
        if po == 0:
            v = jnp.dot(xv[:, 0:84], w1t[...],
                        preferred_element_type=jnp.float32)
        elif po == 13:
            v = jnp.dot(xv[:, 700:784], w1b[...],
                        preferred_element_type=jnp.float32)
        else:
            v = jnp.dot(xv[:, 56 * po - 28:56 * po + 84], w1m[...],
                        preferred_element_type=jnp.float32)
        m = jnp.maximum(jnp.maximum(v[:, 0:128], v[:, 128:256]),
                        jnp.maximum(v[:, 256:384], v[:, 384:512]))
        rows[po + 1] = jnp.maximum(m + b1v, 0.0).astype(jnp.bfloat16)

    b2v = b2[...]
    h1 = jnp.zeros((tb, 128), jnp.float32)
    for po2 in range(7):
        acc = None
        for r in range(4):
            ri = 2 * po2 + r
            if ri in (0, 15):
                continue
            t = jnp.dot(rows[ri], w2[r], preferred_element_type=jnp.float32)
            acc = t if acc is None else acc + t
        m = jnp.maximum(jnp.maximum(acc[:, 0:128], acc[:, 128:256]),
                        jnp.maximum(acc[:, 256:384], acc[:, 384:512]))
        row2 = jnp.maximum(m + b2v, 0.0).astype(jnp.bfloat16)
        h1 = h1 + jnp.dot(row2, f1[po2], preferred_element_type=jnp.float32)

    h1 = jnp.maximum(h1 + fb1[...], 0.0).astype(jnp.bfloat16)
    h2 = jnp.maximum(
        jnp.dot(h1, fw2[...], preferred_element_type=jnp.float32) + fb2[...],
        0.0).astype(jnp.bfloat16)
    logits = jnp.dot(h2, fw3[...], preferred_element_type=jnp.float32) + fb3[...]
    out[...] = logits[:, :10]


def kernel(x, conv1_w, conv1_b, conv2_w, conv2_b,
           fc_w1, fc_b1, fc_w2, fc_b2, fc_w3, fc_b3):
    B = x.shape[0]
    tb = _TB if B % _TB == 0 else B
    grid = B // tb

    w1t, w1m, w1b = _build_conv1_weights(conv1_w)
    W2 = _build_conv2_weights(conv2_w)
    F1 = jnp.pad(fc_w1.reshape(7, 112, 128), ((0, 0), (0, 16), (0, 0)))
    b1v = jnp.concatenate([jnp.zeros((6,), jnp.float32),
                           jnp.tile(conv1_b[0, :6], 14),
                           jnp.zeros((38,), jnp.float32)])[None, :]
    b2v = jnp.concatenate([jnp.tile(conv2_b[0, :16], 7),
                           jnp.zeros((16,), jnp.float32)])[None, :]

    x2d = x.reshape(B, 784).astype(jnp.bfloat16)

    res = pl.pallas_call(
        _net_kernel,
        out_shape=jax.ShapeDtypeStruct((B, 10), jnp.float32),
        grid=(grid,),
        in_specs=[
            pl.BlockSpec((tb, 784), lambda i: (i, 0)),
            pl.BlockSpec((84, 512), lambda i: (0, 0)),
            pl.BlockSpec((112, 512), lambda i: (0, 0)),
            pl.BlockSpec((84, 512), lambda i: (0, 0)),
            pl.BlockSpec((4, 128, 512), lambda i: (0, 0, 0)),
            pl.BlockSpec((7, 128, 128), lambda i: (0, 0, 0)),
            pl.BlockSpec((128, 128), lambda i: (0, 0)),
            pl.BlockSpec((128, 128), lambda i: (0, 0)),
            pl.BlockSpec((1, 128), lambda i: (0, 0)),
            pl.BlockSpec((1, 128), lambda i: (0, 0)),
            pl.BlockSpec((1, 128), lambda i: (0, 0)),
            pl.BlockSpec((1, 128), lambda i: (0, 0)),
            pl.BlockSpec((1, 128), lambda i: (0, 0)),
        ],
        out_specs=pl.BlockSpec((tb, 10), lambda i: (i, 0)),
        compiler_params=pltpu.CompilerParams(
            dimension_semantics=("parallel",)),
    )(x2d, w1t, w1m, w1b, W2, F1, fc_w2, fc_w3, b1v, b2v,
      fc_b1, fc_b2, fc_b3)

    return res


# speedup vs baseline: 132.7061x; 1.2818x over previous
import jax
import jax.numpy as jnp
from jax.experimental import pallas as pl
from jax.experimental.pallas import tpu as pltpu

LANES = 128
_TB = 256


def _build_conv1_weights(conv1_w):
    dt = conv1_w.dtype
    f = conv1_w[:, :6]

    def z(n, m):
        return jnp.zeros((n, m), dt)

    phase_blocks = []
    for ph in range(4):
        di, dj = divmod(ph, 2)
        lead = di * 30 + dj
        E = jnp.concatenate(
            [z(lead, 6), f[0:3], z(27, 6), f[3:6], z(27, 6), f[6:9],
             z(128 - lead - 63, 6)], axis=0)
        cols = [z(128, 6)]
        for q in range(14):
            cols.append(jnp.concatenate([z(2 * q, 6), E[:128 - 2 * q]], axis=0))
        cols.append(z(128, 38))
        phase_blocks.append(jnp.concatenate(cols, axis=1))
    W1big = jnp.concatenate(phase_blocks, axis=1)

    W1r = W1big[:120].reshape(4, 30, 512)
    w1mid = W1r[:, 1:29].reshape(112, 512)
    w1top = W1r[1:4, 1:29].reshape(84, 512)
    w1bot = W1r[0:3, 1:29].reshape(84, 512)
    return w1top, w1mid, w1bot


def _build_conv2_weights(conv2_w):
    dt = conv2_w.dtype

    def z(n, m):
        return jnp.zeros((n, m), dt)

    per_r = []
    for r in range(4):
        phase_blocks = []
        for ph in range(4):
            di, dj = divmod(ph, 2)
            ki = r - di
            if 0 <= ki <= 2:
                sl = conv2_w[ki * 18:ki * 18 + 18, :16]
                E = jnp.concatenate([z(6 * dj, 16), sl, z(6 - 6 * dj, 16)],
                                    axis=0)
            else:
                E = z(24, 16)
            E128 = jnp.concatenate([E, z(104, 16)], axis=0)
            cols = []
            for q2 in range(7):
                cols.append(jnp.concatenate(
                    [z(12 * q2, 16), E128[:128 - 12 * q2]], axis=0))
            cols.append(z(128, 16))
            phase_blocks.append(jnp.concatenate(cols, axis=1))
        per_r.append(jnp.concatenate(phase_blocks, axis=1))
    return jnp.stack(per_r, axis=0)


def _net_kernel(x_ref, w1t, w1m, w1b, w2, f1, fw2, fw3,
                b1, b2, fb1, fb2, fb3, out):
    tb = out.shape[0]
    xv = x_ref[...].reshape(tb, 784).astype(jnp.bfloat16)
    b1v = b1[...]

    rows = [None] * 16
    zero = jnp.zeros((tb, 128), jnp.bfloat16)
    rows[0] = zero
    rows[15] = zero
    for po in range(14):
        if po == 0:
            v = jnp.dot(xv[:, 0:84], w1t[...],
                        preferred_element_type=jnp.float32)
        elif po == 13:
            v = jnp.dot(xv[:, 700:784], w1b[...],
                        preferred_element_type=jnp.float32)
        else:
            v = jnp.dot(xv[:, 56 * po - 28:56 * po + 84], w1m[...],
                        preferred_element_type=jnp.float32)
        m = jnp.maximum(jnp.maximum(v[:, 0:128], v[:, 128:256]),
                        jnp.maximum(v[:, 256:384], v[:, 384:512]))
        rows[po + 1] = jnp.maximum(m + b1v, 0.0).astype(jnp.bfloat16)

    b2v = b2[...]
    h1 = jnp.zeros((tb, 128), jnp.float32)
    for po2 in range(7):
        acc = None
        for r in range(4):
            ri = 2 * po2 + r
            if ri in (0, 15):
                continue
            t = jnp.dot(rows[ri], w2[r], preferred_element_type=jnp.float32)
            acc = t if acc is None else acc + t
        m = jnp.maximum(jnp.maximum(acc[:, 0:128], acc[:, 128:256]),
                        jnp.maximum(acc[:, 256:384], acc[:, 384:512]))
        row2 = jnp.maximum(m + b2v, 0.0).astype(jnp.bfloat16)
        h1 = h1 + jnp.dot(row2, f1[po2], preferred_element_type=jnp.float32)

    h1 = jnp.maximum(h1 + fb1[...], 0.0).astype(jnp.bfloat16)
    h2 = jnp.maximum(
        jnp.dot(h1, fw2[...], preferred_element_type=jnp.float32) + fb2[...],
        0.0).astype(jnp.bfloat16)
    logits = jnp.dot(h2, fw3[...], preferred_element_type=jnp.float32) + fb3[...]
    out[...] = logits[:, :10]


def kernel(x, conv1_w, conv1_b, conv2_w, conv2_b,
           fc_w1, fc_b1, fc_w2, fc_b2, fc_w3, fc_b3):
    B = x.shape[0]
    tb = _TB if B % _TB == 0 else B
    grid = B // tb

    w1t, w1m, w1b = _build_conv1_weights(conv1_w)
    W2 = _build_conv2_weights(conv2_w)
    F1 = jnp.pad(fc_w1.reshape(7, 112, 128), ((0, 0), (0, 16), (0, 0)))
    b1v = jnp.concatenate([jnp.zeros((6,), jnp.float32),
                           jnp.tile(conv1_b[0, :6], 14),
                           jnp.zeros((38,), jnp.float32)])[None, :]
    b2v = jnp.concatenate([jnp.tile(conv2_b[0, :16], 7),
                           jnp.zeros((16,), jnp.float32)])[None, :]

    x3d = x.reshape(B, 28, 28)

    res = pl.pallas_call(
        _net_kernel,
        out_shape=jax.ShapeDtypeStruct((B, 10), jnp.float32),
        grid=(grid,),
        in_specs=[
            pl.BlockSpec((tb, 28, 28), lambda i: (i, 0, 0)),
            pl.BlockSpec((84, 512), lambda i: (0, 0)),
            pl.BlockSpec((112, 512), lambda i: (0, 0)),
            pl.BlockSpec((84, 512), lambda i: (0, 0)),
            pl.BlockSpec((4, 128, 512), lambda i: (0, 0, 0)),
            pl.BlockSpec((7, 128, 128), lambda i: (0, 0, 0)),
            pl.BlockSpec((128, 128), lambda i: (0, 0)),
            pl.BlockSpec((128, 128), lambda i: (0, 0)),
            pl.BlockSpec((1, 128), lambda i: (0, 0)),
            pl.BlockSpec((1, 128), lambda i: (0, 0)),
            pl.BlockSpec((1, 128), lambda i: (0, 0)),
            pl.BlockSpec((1, 128), lambda i: (0, 0)),
            pl.BlockSpec((1, 128), lambda i: (0, 0)),
        ],
        out_specs=pl.BlockSpec((tb, 10), lambda i: (i, 0)),
        compiler_params=pltpu.CompilerParams(
            dimension_semantics=("parallel",)),
    )(x3d, w1t, w1m, w1b, W2, F1, fc_w2, fc_w3, b1v, b2v,
      fc_b1, fc_b2, fc_b3)

    return res


# speedup vs baseline: 177.8957x; 1.3405x over previous
import numpy as np
import jax
import jax.numpy as jnp
from jax.experimental import pallas as pl
from jax.experimental.pallas import tpu as pltpu

LANES = 128
_TB = 512


def _conv1_selectors():
    A = np.zeros((9, 128, 512), np.float32)
    C = np.zeros((6, 512), np.float32)
    for ph in range(4):
        di, dj = divmod(ph, 2)
        for q in range(14):
            for c in range(6):
                n = ph * 128 + (q + 1) * 6 + c
                C[c, n] = 1.0
                for ki in range(3):
                    for kj in range(3):
                        k = (ki + di) * 30 + (kj + dj) + 2 * q
                        A[ki * 3 + kj, k, n] = 1.0
    return A.astype(jnp.bfloat16), C.astype(jnp.bfloat16)


def _conv2_selectors():
    B = np.zeros((9 * 128, 54), np.float32)
    for t in range(9):
        for k in range(96):
            B[t * 128 + k, t * 6 + k % 6] = 1.0
    O = np.zeros((128, 512), np.float32)
    for n in range(512):
        if n % 128 < 112:
            O[n % 16, n] = 1.0
    A = np.zeros((4, 9, 128, 512), np.float32)
    for r in range(4):
        for ph in range(4):
            di, dj = divmod(ph, 2)
            ki = r - di
            if not 0 <= ki <= 2:
                continue
            for q2 in range(7):
                for kj in range(3):
                    cp = 2 * q2 + dj + kj
                    for ch in range(6):
                        k = cp * 6 + ch
                        for o in range(16):
                            A[r, ki * 3 + kj, k, ph * 128 + q2 * 16 + o] = 1.0
    return (B.astype(jnp.bfloat16), O.astype(jnp.bfloat16),
            A.astype(jnp.bfloat16))


_A1, _C1 = _conv1_selectors()
_B2, _O2, _A2 = _conv2_selectors()


def _build_weights(conv1_w, conv2_w):
    G1 = jnp.dot(conv1_w[:, :6], _C1)
    W1big = jnp.einsum("tkn,tn->kn", _A1, G1)
    W1r = W1big[:120].reshape(4, 30, 512)
    w1mid = W1r[:, 1:29].reshape(112, 512)
    w1top = W1r[1:4, 1:29].reshape(84, 512)
    w1bot = W1r[0:3, 1:29].reshape(84, 512)
    z = jnp.zeros((28, 512), conv1_w.dtype)
    w1p0 = jnp.concatenate([
        jnp.concatenate([w1top, jnp.zeros((56, 512), conv1_w.dtype)], 0),
        jnp.concatenate([z, w1mid], 0)], axis=1)
    w1pm = jnp.concatenate([
        jnp.concatenate([w1mid, jnp.zeros((56, 512), conv1_w.dtype)], 0),
        jnp.concatenate([jnp.zeros((56, 512), conv1_w.dtype), w1mid], 0)],
        axis=1)
    w1p6 = jnp.concatenate([
        jnp.concatenate([w1mid, z], 0),
        jnp.concatenate([jnp.zeros((56, 512), conv1_w.dtype), w1bot], 0)],
        axis=1)

    Q = jnp.dot(_B2, conv2_w)
    Qn = jnp.dot(Q, _O2).reshape(9, 128, 512)
    W2 = jnp.einsum("rtkn,tkn->rkn", _A2, Qn)
    W2a = W2[0:2].reshape(256, 512)
    W2b = W2[2:4].reshape(256, 512)
    return w1p0, w1pm, w1p6, W2a, W2b


def _net_kernel(x_ref, w1p0, w1pm, w1p6, w2a, w2b, f1, fw2, fw3,
                b1, b2, fb1, fb2, fb3, out):
    tb = out.shape[0]
    xv = x_ref[...].reshape(tb, 784).astype(jnp.bfloat16)
    b1v = b1[...]

    rows = [None] * 16
    zero = jnp.zeros((tb, 128), jnp.bfloat16)
    rows[0] = zero
    rows[15] = zero
    for j in range(7):
        po = 2 * j
        if j == 0:
            v = jnp.dot(xv[:, 0:140], w1p0[...],
                        preferred_element_type=jnp.float32)
        elif j == 6:
            v = jnp.dot(xv[:, 644:784], w1p6[...],
                        preferred_element_type=jnp.float32)
        else:
            v = jnp.dot(xv[:, 56 * po - 28:56 * po + 140], w1pm[...],
                        preferred_element_type=jnp.float32)
        for s in range(2):
            b = 512 * s
            m = jnp.maximum(
                jnp.maximum(v[:, b:b + 128], v[:, b + 128:b + 256]),
                jnp.maximum(v[:, b + 256:b + 384], v[:, b + 384:b + 512]))
            rows[po + s + 1] = jnp.maximum(m + b1v, 0.0).astype(jnp.bfloat16)

    pairs = [jnp.concatenate([rows[2 * j], rows[2 * j + 1]], axis=1)
             for j in range(8)]

    b2v = b2[...]
    row2s = []
    for po2 in range(7):
        acc = (jnp.dot(pairs[po2], w2a[...], preferred_element_type=jnp.float32)
               + jnp.dot(pairs[po2 + 1], w2b[...],
                         preferred_element_type=jnp.float32))
        m = jnp.maximum(jnp.maximum(acc[:, 0:128], acc[:, 128:256]),
                        jnp.maximum(acc[:, 256:384], acc[:, 384:512]))
        row2s.append(jnp.maximum(m + b2v, 0.0).astype(jnp.bfloat16))

    h1 = None
    for j in range(3):
        op = jnp.concatenate([row2s[2 * j], row2s[2 * j + 1]], axis=1)
        t = jnp.dot(op, f1[j], preferred_element_type=jnp.float32)
        h1 = t if h1 is None else h1 + t
    h1 = h1 + jnp.dot(row2s[6], f1[3][:128],
                      preferred_element_type=jnp.float32)

    h1 = jnp.maximum(h1 + fb1[...], 0.0).astype(jnp.bfloat16)
    h2 = jnp.maximum(
        jnp.dot(h1, fw2[...], preferred_element_type=jnp.float32) + fb2[...],
        0.0).astype(jnp.bfloat16)
    logits = jnp.dot(h2, fw3[...], preferred_element_type=jnp.float32) + fb3[...]
    out[...] = logits[:, :10]


def kernel(x, conv1_w, conv1_b, conv2_w, conv2_b,
           fc_w1, fc_b1, fc_w2, fc_b2, fc_w3, fc_b3):
    B = x.shape[0]
    tb = _TB if B % _TB == 0 else B
    grid = B // tb

    w1p0, w1pm, w1p6, W2a, W2b = _build_weights(conv1_w, conv2_w)
    F1r = jnp.pad(fc_w1.reshape(7, 112, 128), ((0, 1), (0, 16), (0, 0)))
    F1 = F1r.reshape(4, 256, 128)
    b1v = jnp.concatenate([jnp.zeros((6,), jnp.float32),
                           jnp.tile(conv1_b[0, :6], 14),
                           jnp.zeros((38,), jnp.float32)])[None, :]
    b2v = jnp.concatenate([jnp.tile(conv2_b[0, :16], 7),
                           jnp.zeros((16,), jnp.float32)])[None, :]

    x3d = x.reshape(B, 28, 28)

    res = pl.pallas_call(
        _net_kernel,
        out_shape=jax.ShapeDtypeStruct((B, 10), jnp.float32),
        grid=(grid,),
        in_specs=[
            pl.BlockSpec((tb, 28, 28), lambda i: (i, 0, 0)),
            pl.BlockSpec((140, 1024), lambda i: (0, 0)),
            pl.BlockSpec((168, 1024), lambda i: (0, 0)),
            pl.BlockSpec((140, 1024), lambda i: (0, 0)),
            pl.BlockSpec((256, 512), lambda i: (0, 0)),
            pl.BlockSpec((256, 512), lambda i: (0, 0)),
            pl.BlockSpec((4, 256, 128), lambda i: (0, 0, 0)),
            pl.BlockSpec((128, 128), lambda i: (0, 0)),
            pl.BlockSpec((128, 128), lambda i: (0, 0)),
            pl.BlockSpec((1, 128), lambda i: (0, 0)),
            pl.BlockSpec((1, 128), lambda i: (0, 0)),
            pl.BlockSpec((1, 128), lambda i: (0, 0)),
            pl.BlockSpec((1, 128), lambda i: (0, 0)),
            pl.BlockSpec((1, 128), lambda i: (0, 0)),
        ],
        out_specs=pl.BlockSpec((tb, 10), lambda i: (i, 0)),
        compiler_params=pltpu.CompilerParams(
            dimension_semantics=("parallel",)),
    )(x3d, w1p0, w1pm, w1p6, W2a, W2b, F1, fc_w2, fc_w3, b1v, b2v,
      fc_b1, fc_b2, fc_b3)

    return res


# speedup vs baseline: 177.9594x; 1.0004x over previous
import numpy as np
import jax
import jax.numpy as jnp
from jax.experimental import pallas as pl
from jax.experimental.pallas import tpu as pltpu

LANES = 128
_TB = 512


def _conv1_sel():
    S1 = np.full((128, 512), -1, np.int64)
    for ph in range(4):
        di, dj = divmod(ph, 2)
        for q in range(14):
            for c in range(6):
                n0 = ph * 128 + (q + 1) * 6 + c
                for ki in range(3):
                    for kj in range(3):
                        k = (ki + di) * 30 + (kj + dj) + 2 * q
                        S1[k, n0] = ki * 3 + kj
    mid = np.array([[S1[r * 30 + j + 1] for j in range(28)]
                    for r in range(4)]).reshape(112, 512)
    top = np.array([[S1[(r + 1) * 30 + j + 1] for j in range(28)]
                    for r in range(3)]).reshape(84, 512)
    bot = np.array([[S1[r * 30 + j + 1] for j in range(28)]
                    for r in range(3)]).reshape(84, 512)
    sel = np.full((3, 176, 1024), -1, np.int64)
    sel[0, 0:84, 0:512] = top
    sel[0, 28:140, 512:1024] = mid
    sel[1, 0:112, 0:512] = mid
    sel[1, 56:168, 512:1024] = mid
    sel[2, 36:148, 0:512] = mid
    sel[2, 92:176, 512:1024] = bot
    A = (sel[None] == np.arange(9)[:, None, None, None])
    C = np.zeros((6, 1024), np.float32)
    for half in range(2):
        for ph in range(4):
            for q in range(14):
                for c in range(6):
                    C[c, half * 512 + ph * 128 + (q + 1) * 6 + c] = 1.0
    return (jnp.asarray(A.astype(np.float32), jnp.bfloat16),
            jnp.asarray(C, jnp.bfloat16))


def _conv2_sel():
    B = np.zeros((9 * 256, 54), np.float32)
    for t in range(9):
        for h in range(2):
            for k in range(96):
                B[t * 256 + h * 128 + k, t * 6 + k % 6] = 1.0
    O = np.zeros((128, 512), np.float32)
    for n in range(512):
        if n % 128 < 112:
            O[n % 16, n] = 1.0
    A = np.zeros((9, 2, 256, 512), np.float32)
    for r in range(4):
        p, h = divmod(r, 2)
        for ph in range(4):
            di, dj = divmod(ph, 2)
            ki = r - di
            if not 0 <= ki <= 2:
                continue
            for q2 in range(7):
                for kj in range(3):
                    cp = 2 * q2 + dj + kj
                    for ch in range(6):
                        kk = h * 128 + cp * 6 + ch
                        n0 = ph * 128 + q2 * 16
                        A[ki * 3 + kj, p, kk, n0:n0 + 16] = 1.0
    return (jnp.asarray(B, jnp.bfloat16), jnp.asarray(O, jnp.bfloat16),
            jnp.asarray(A, jnp.bfloat16))


_A1, _C1 = _conv1_sel()
_B2, _O2, _A2 = _conv2_sel()


def _net_kernel(x_ref, w1, w2, f1, fw2, fw3, b1, b2, fb1, fb2, fb3, out):
    tb = out.shape[0]
    xv = x_ref[...].reshape(tb, 784).astype(jnp.bfloat16)
    b1v = b1[...]

    rows = [None] * 16
    zero = jnp.zeros((tb, 128), jnp.bfloat16)
    rows[0] = zero
    rows[15] = zero
    for j in range(7):
        po = 2 * j
        if j == 0:
            v = jnp.dot(xv[:, 0:176], w1[0],
                        preferred_element_type=jnp.float32)
        elif j == 6:
            v = jnp.dot(xv[:, 608:784], w1[2],
                        preferred_element_type=jnp.float32)
        else:
            v = jnp.dot(xv[:, 56 * po - 28:56 * po + 148], w1[1],
                        preferred_element_type=jnp.float32)
        for s in range(2):
            b = 512 * s
            m = jnp.maximum(
                jnp.maximum(v[:, b:b + 128], v[:, b + 128:b + 256]),
                jnp.maximum(v[:, b + 256:b + 384], v[:, b + 384:b + 512]))
            rows[po + s + 1] = jnp.maximum(m + b1v, 0.0).astype(jnp.bfloat16)

    pairs = [jnp.concatenate([rows[2 * j], rows[2 * j + 1]], axis=1)
             for j in range(8)]

    b2v = b2[...]
    row2s = []
    for po2 in range(7):
        acc = (jnp.dot(pairs[po2], w2[0], preferred_element_type=jnp.float32)
               + jnp.dot(pairs[po2 + 1], w2[1],
                         preferred_element_type=jnp.float32))
        m = jnp.maximum(jnp.maximum(acc[:, 0:128], acc[:, 128:256]),
                        jnp.maximum(acc[:, 256:384], acc[:, 384:512]))
        row2s.append(jnp.maximum(m + b2v, 0.0).astype(jnp.bfloat16))

    h1 = None
    for j in range(3):
        op = jnp.concatenate([row2s[2 * j], row2s[2 * j + 1]], axis=1)
        t = jnp.dot(op, f1[j], preferred_element_type=jnp.float32)
        h1 = t if h1 is None else h1 + t
    h1 = h1 + jnp.dot(row2s[6], f1[3][:128],
                      preferred_element_type=jnp.float32)

    h1 = jnp.maximum(h1 + fb1[...], 0.0).astype(jnp.bfloat16)
    h2 = jnp.maximum(
        jnp.dot(h1, fw2[...], preferred_element_type=jnp.float32) + fb2[...],
        0.0).astype(jnp.bfloat16)
    logits = jnp.dot(h2, fw3[...], preferred_element_type=jnp.float32) + fb3[...]
    out[...] = logits[:, :10]


def kernel(x, conv1_w, conv1_b, conv2_w, conv2_b,
           fc_w1, fc_b1, fc_w2, fc_b2, fc_w3, fc_b3):
    B = x.shape[0]
    tb = _TB if B % _TB == 0 else B
    grid = B // tb

    G1 = jnp.dot(conv1_w[:, :6], _C1)
    w1all = jnp.einsum("tjkn,tn->jkn", _A1, G1)
    Qn = jnp.dot(jnp.dot(_B2, conv2_w), _O2).reshape(9, 256, 512)
    w2ab = jnp.einsum("tpkn,tkn->pkn", _A2, Qn)

    F1 = jnp.pad(fc_w1.reshape(7, 112, 128),
                 ((0, 1), (0, 16), (0, 0))).reshape(4, 256, 128)
    b1v = jnp.concatenate([jnp.zeros((6,), jnp.float32),
                           jnp.tile(conv1_b[0, :6], 14),
                           jnp.zeros((38,), jnp.float32)])[None, :]
    b2v = jnp.concatenate([jnp.tile(conv2_b[0, :16], 7),
                           jnp.zeros((16,), jnp.float32)])[None, :]

    x3d = x.reshape(B, 28, 28)

    res = pl.pallas_call(
        _net_kernel,
        out_shape=jax.ShapeDtypeStruct((B, 10), jnp.float32),
        grid=(grid,),
        in_specs=[
            pl.BlockSpec((tb, 28, 28), lambda i: (i, 0, 0)),
            pl.BlockSpec((3, 176, 1024), lambda i: (0, 0, 0)),
            pl.BlockSpec((2, 256, 512), lambda i: (0, 0, 0)),
            pl.BlockSpec((4, 256, 128), lambda i: (0, 0, 0)),
            pl.BlockSpec((128, 128), lambda i: (0, 0)),
            pl.BlockSpec((128, 128), lambda i: (0, 0)),
            pl.BlockSpec((1, 128), lambda i: (0, 0)),
            pl.BlockSpec((1, 128), lambda i: (0, 0)),
            pl.BlockSpec((1, 128), lambda i: (0, 0)),
            pl.BlockSpec((1, 128), lambda i: (0, 0)),
            pl.BlockSpec((1, 128), lambda i: (0, 0)),
        ],
        out_specs=pl.BlockSpec((tb, 10), lambda i: (i, 0)),
        compiler_params=pltpu.CompilerParams(
            dimension_semantics=("parallel",)),
    )(x3d, w1all, w2ab, F1, fc_w2, fc_w3, b1v, b2v,
      fc_b1, fc_b2, fc_b3)

    return res
